# Initial kernel scaffold; baseline (speedup 1.0000x reference)
import jax
import jax.numpy as jnp
from jax.experimental import pallas as pl


def kernel(x, W_in, b_in, W_mid0, b_mid0, W_mid1, b_mid1, W_out, b_out, indx_seqs):
    raise NotImplementedError("write your pallas kernel here")


# speedup vs baseline: 2.9509x; 2.9509x over previous
import functools

import jax
import jax.numpy as jnp
from jax import lax
from jax.experimental import pallas as pl

NUM_IN = 1024
NUM_OUT = 1024
NUM_MID = 4096
FAN_IN = 16
BATCH = 1024

TJ = 512


def _mm_in_body(x_ref, w_ref, b_ref, o_ref):
    acc = lax.dot_general(
        x_ref[...], w_ref[...], (((1,), (1,)), ((), ())),
        preferred_element_type=jnp.float32)
    o_ref[...] = jnp.maximum(acc + b_ref[...].reshape(1, -1), 0.0)


def _cond_body(a_ref, w_ref, b_ref, idx_ref, o_ref):
    idx = idx_ref[...]
    w = w_ref[...]
    iota_m = lax.broadcasted_iota(jnp.int32, (NUM_MID, TJ), 0)
    s_t = jnp.zeros((NUM_MID, TJ), jnp.float32)
    for k in range(FAN_IN):
        s_t = s_t + jnp.where(iota_m == idx[:, k].reshape(1, TJ),
                              w[:, k].reshape(1, TJ), 0.0)
    acc = lax.dot_general(
        a_ref[...], s_t, (((1,), (0,)), ((), ())),
        preferred_element_type=jnp.float32)
    o_ref[...] = jnp.maximum(acc + b_ref[...].reshape(1, -1), 0.0)


def _mm_out_body(a_ref, w_ref, b_ref, o_ref):
    acc = lax.dot_general(
        a_ref[...], w_ref[...], (((1,), (1,)), ((), ())),
        preferred_element_type=jnp.float32)
    o_ref[...] = acc + b_ref[...].reshape(1, -1)


def _mm_in(x, W_in, b_in):
    grid = (NUM_MID // TJ,)
    return pl.pallas_call(
        _mm_in_body,
        grid=grid,
        in_specs=[
            pl.BlockSpec((BATCH, NUM_IN), lambda i: (0, 0)),
            pl.BlockSpec((TJ, NUM_IN), lambda i: (i, 0)),
            pl.BlockSpec((1, TJ), lambda i: (0, i)),
        ],
        out_specs=pl.BlockSpec((BATCH, TJ), lambda i: (0, i)),
        out_shape=jax.ShapeDtypeStruct((BATCH, NUM_MID), jnp.float32),
    )(x, W_in, b_in.reshape(1, NUM_MID))


def _cond(a, W, b, idx):
    grid = (NUM_MID // TJ,)
    return pl.pallas_call(
        _cond_body,
        grid=grid,
        in_specs=[
            pl.BlockSpec((BATCH, NUM_MID), lambda i: (0, 0)),
            pl.BlockSpec((TJ, FAN_IN), lambda i: (i, 0)),
            pl.BlockSpec((1, TJ), lambda i: (0, i)),
            pl.BlockSpec((TJ, FAN_IN), lambda i: (i, 0)),
        ],
        out_specs=pl.BlockSpec((BATCH, TJ), lambda i: (0, i)),
        out_shape=jax.ShapeDtypeStruct((BATCH, NUM_MID), jnp.float32),
    )(a, W, b.reshape(1, NUM_MID), idx)


def _mm_out(a, W_out, b_out):
    grid = (NUM_OUT // TJ,)
    return pl.pallas_call(
        _mm_out_body,
        grid=grid,
        in_specs=[
            pl.BlockSpec((BATCH, NUM_MID), lambda i: (0, 0)),
            pl.BlockSpec((TJ, NUM_MID), lambda i: (i, 0)),
            pl.BlockSpec((1, TJ), lambda i: (0, i)),
        ],
        out_specs=pl.BlockSpec((BATCH, TJ), lambda i: (0, i)),
        out_shape=jax.ShapeDtypeStruct((BATCH, NUM_OUT), jnp.float32),
    )(a, W_out, b_out.reshape(1, NUM_OUT))


@jax.jit
def kernel(x, W_in, b_in, W_mid0, b_mid0, W_mid1, b_mid1, W_out, b_out,
           indx_seqs):
    a = _mm_in(x, W_in, b_in)
    a = _cond(a, W_mid0, b_mid0, indx_seqs)
    a = _cond(a, W_mid1, b_mid1, indx_seqs)
    return _mm_out(a, W_out, b_out)
